# baseline (speedup 1.0000x reference)
import functools

import jax
import jax.numpy as jnp
from jax import lax
from jax.experimental import pallas as pl
from jax.experimental.pallas import tpu as pltpu
from jax.experimental.pallas import tpu_sc as plsc

BS = 32
P = 196
D = 32
K = 512
PS = 16
NROW = 14

NC = 2
NS = 16
NW = NC * NS
IMG = NROW * PS


def _assign_body(img_ref, cent_ref, out_ref):
    x = img_ref[0]
    c = cent_ref[...]
    dot = jnp.dot(x, c, preferred_element_type=jnp.float32,
                  precision=lax.Precision.HIGHEST)
    cn = jnp.sum(c * c, axis=0, keepdims=True)
    score = cn - 2.0 * dot
    m = jnp.max(score, axis=1, keepdims=True)
    ids = lax.broadcasted_iota(jnp.int32, (P, K), 1)
    a = jnp.min(jnp.where(score >= m, ids, K), axis=1)
    out_ref[...] = a.reshape(1, 1, P)


def _assignment(image, centroids):
    return pl.pallas_call(
        _assign_body,
        grid=(BS,),
        in_specs=[
            pl.BlockSpec((1, P, D), lambda b: (b, 0, 0)),
            pl.BlockSpec((D, K), lambda b: (0, 0)),
        ],
        out_specs=pl.BlockSpec((1, 1, P), lambda b: (b, 0, 0)),
        out_shape=jax.ShapeDtypeStruct((BS, 1, P), jnp.int32),
    )(image, centroids)


@functools.cache
def _sc_gather_kernel():
    mesh = plsc.VectorSubcoreMesh(core_axis_name="c", subcore_axis_name="s")

    @functools.partial(
        pl.kernel,
        mesh=mesh,
        out_type=jax.ShapeDtypeStruct((BS, 112, 128), jnp.int32),
        compiler_params=pltpu.CompilerParams(needs_layout_passes=False),
        scratch_types=[
            pltpu.VMEM_SHARED((K // 2, 128), jnp.int32),
            pltpu.VMEM((K // 2, 128), jnp.int32),
            pltpu.VMEM((2, 128), jnp.int32),
            pltpu.VMEM((112, 128), jnp.int32),
            pltpu.SemaphoreType.DMA,
        ],
    )
    def _sc_gather(table_hbm, assign_hbm, out_hbm, tab_sh, tab_v, a_v, out_v,
                   sem):
        cid = lax.axis_index("c")
        sid = lax.axis_index("s")
        wid = sid * NC + cid

        @pl.when(sid == 0)
        def _stage():
            pltpu.sync_copy(table_hbm, tab_sh)

        plsc.subcore_barrier()
        pltpu.sync_copy(tab_sh, tab_v)
        pltpu.sync_copy(assign_hbm.at[wid], a_v)

        lanes = lax.iota(jnp.int32, PS)
        kvec = lanes & 3

        def body(r, carry):
            for g in range(4):
                pvec = r * PS + g * 4 + (lanes >> 2)
                a_lane = plsc.load_gather(a_v, [pvec >> 7, pvec & 127])
                trow = a_lane >> 1
                tbase = (a_lane & 1) * 64
                for i in range(PS):
                    words = plsc.load_gather(
                        tab_v, [trow, tbase + i * 4 + kvec])
                    out_v[8 * r + (4 * i + g) // 8,
                          pl.ds(64 * (i % 2) + 16 * g, PS)] = words
            return carry

        lax.fori_loop(0, NROW, body, 0)
        pltpu.sync_copy(out_v, out_hbm.at[wid])

    return _sc_gather


def kernel(image, centroids, cluster_labels):
    assign = _assignment(image, centroids).reshape(BS, NROW, NROW)
    assign = jnp.pad(assign, ((0, 0), (0, 0), (0, PS - NROW)))
    assign = jnp.pad(assign.reshape(BS, IMG), ((0, 0), (0, 256 - IMG)))
    assign = assign.reshape(BS, 2, 128)
    t8 = jnp.transpose(cluster_labels).astype(jnp.int8)
    table32 = lax.bitcast_convert_type(t8.reshape(K, 64, 4), jnp.int32)
    table32 = table32.reshape(K // 2, 128)
    out32 = _sc_gather_kernel()(table32, assign)
    b8 = lax.bitcast_convert_type(out32, jnp.int8)
    pred = b8.reshape(BS, IMG, 256)[:, :, :IMG].astype(jnp.int32)
    return pred


# speedup vs baseline: 1.0006x; 1.0006x over previous
import functools

import jax
import jax.numpy as jnp
from jax import lax
from jax.experimental import pallas as pl
from jax.experimental.pallas import tpu as pltpu
from jax.experimental.pallas import tpu_sc as plsc

BS = 32
P = 196
D = 32
K = 512
PS = 16
NROW = 14

NC = 2
NS = 16
NW = NC * NS
IMG = NROW * PS


def _assign_body(img_ref, cent_ref, out_ref):
    x = img_ref[0]
    c = cent_ref[...]
    dot = jnp.dot(x, c, preferred_element_type=jnp.float32,
                  precision=lax.Precision.HIGHEST)
    cn = jnp.sum(c * c, axis=0, keepdims=True)
    score = cn - 2.0 * dot
    m = jnp.max(score, axis=1, keepdims=True)
    ids = lax.broadcasted_iota(jnp.int32, (P, K), 1)
    a = jnp.min(jnp.where(score >= m, ids, K), axis=1)
    out_ref[...] = a.reshape(1, 1, P)


def _assignment(image, centroids):
    return pl.pallas_call(
        _assign_body,
        grid=(BS,),
        in_specs=[
            pl.BlockSpec((1, P, D), lambda b: (b, 0, 0)),
            pl.BlockSpec((D, K), lambda b: (0, 0)),
        ],
        out_specs=pl.BlockSpec((1, 1, P), lambda b: (b, 0, 0)),
        out_shape=jax.ShapeDtypeStruct((BS, 1, P), jnp.int32),
    )(image, centroids)


@functools.cache
def _sc_gather_kernel():
    mesh = plsc.VectorSubcoreMesh(core_axis_name="c", subcore_axis_name="s")

    @functools.partial(
        pl.kernel,
        mesh=mesh,
        out_type=jax.ShapeDtypeStruct((BS, 112, 128), jnp.int32),
        compiler_params=pltpu.CompilerParams(needs_layout_passes=False),
        scratch_types=[
            pltpu.VMEM_SHARED((K // 2, 128), jnp.int32),
            pltpu.VMEM((K // 2, 128), jnp.int32),
            pltpu.VMEM((2, 128), jnp.int32),
            pltpu.VMEM((112, 128), jnp.int32),
            pltpu.SemaphoreType.DMA,
        ],
    )
    def _sc_gather(table_hbm, assign_hbm, out_hbm, tab_sh, tab_v, a_v, out_v,
                   sem):
        cid = lax.axis_index("c")
        sid = lax.axis_index("s")
        wid = sid * NC + cid

        with jax.named_scope("stage_table"):
            @pl.when(sid == 0)
            def _stage():
                pltpu.sync_copy(table_hbm, tab_sh)

            plsc.subcore_barrier()
            pltpu.sync_copy(tab_sh, tab_v)
            pltpu.sync_copy(assign_hbm.at[wid], a_v)

        lanes = lax.iota(jnp.int32, PS)
        kvec = lanes & 3

        def body(r, carry):
            for g in range(4):
                pvec = r * PS + g * 4 + (lanes >> 2)
                a_lane = plsc.load_gather(a_v, [pvec >> 7, pvec & 127])
                trow = a_lane >> 1
                tbase = (a_lane & 1) * 64
                for i in range(PS):
                    words = plsc.load_gather(
                        tab_v, [trow, tbase + i * 4 + kvec])
                    out_v[8 * r + (4 * i + g) // 8,
                          pl.ds(64 * (i % 2) + 16 * g, PS)] = words
            return carry

        with jax.named_scope("assemble"):
            lax.fori_loop(0, NROW, body, 0)
        with jax.named_scope("writeback"):
            pltpu.sync_copy(out_v, out_hbm.at[wid])

    return _sc_gather


def kernel(image, centroids, cluster_labels):
    assign = _assignment(image, centroids).reshape(BS, NROW, NROW)
    assign = jnp.pad(assign, ((0, 0), (0, 0), (0, PS - NROW)))
    assign = jnp.pad(assign.reshape(BS, IMG), ((0, 0), (0, 256 - IMG)))
    assign = assign.reshape(BS, 2, 128)
    t8 = jnp.transpose(cluster_labels).astype(jnp.int8)
    table32 = lax.bitcast_convert_type(t8.reshape(K, 64, 4), jnp.int32)
    table32 = table32.reshape(K // 2, 128)
    out32 = _sc_gather_kernel()(table32, assign)
    b8 = lax.bitcast_convert_type(out32, jnp.int8)
    pred = b8.reshape(BS, IMG, 256)[:, :, :IMG].astype(jnp.int32)
    return pred


# speedup vs baseline: 1.1135x; 1.1128x over previous
import functools

import jax
import jax.numpy as jnp
from jax import lax
from jax.experimental import pallas as pl
from jax.experimental.pallas import tpu as pltpu
from jax.experimental.pallas import tpu_sc as plsc

BS = 32
P = 196
D = 32
K = 512
PS = 16
NROW = 14

NC = 2
NS = 16
NW = NC * NS
IMG = NROW * PS


def _assign_body(img_ref, cent_ref, out_ref):
    x = img_ref[0]
    c = cent_ref[...]
    dot = jnp.dot(x, c, preferred_element_type=jnp.float32,
                  precision=lax.Precision.HIGHEST)
    cn = jnp.sum(c * c, axis=0, keepdims=True)
    score = cn - 2.0 * dot
    m = jnp.max(score, axis=1, keepdims=True)
    ids = lax.broadcasted_iota(jnp.int32, (P, K), 1)
    a = jnp.min(jnp.where(score >= m, ids, K), axis=1)
    ap = jnp.concatenate([a, jnp.zeros((256 - P,), jnp.int32)])
    out_ref[...] = ap.reshape(1, 2, 128)


def _assignment(image, centroids):
    return pl.pallas_call(
        _assign_body,
        grid=(BS,),
        in_specs=[
            pl.BlockSpec((1, P, D), lambda b: (b, 0, 0)),
            pl.BlockSpec((D, K), lambda b: (0, 0)),
        ],
        out_specs=pl.BlockSpec((1, 2, 128), lambda b: (b, 0, 0)),
        out_shape=jax.ShapeDtypeStruct((BS, 2, 128), jnp.int32),
    )(image, centroids)


@functools.cache
def _sc_gather_kernel():
    mesh = plsc.VectorSubcoreMesh(core_axis_name="c", subcore_axis_name="s")

    @functools.partial(
        pl.kernel,
        mesh=mesh,
        out_type=jax.ShapeDtypeStruct((BS, 112, 128), jnp.int32),
        compiler_params=pltpu.CompilerParams(needs_layout_passes=False),
        scratch_types=[
            pltpu.VMEM_SHARED((K // 2, 128), jnp.int32),
            pltpu.VMEM((K // 2, 128), jnp.int32),
            pltpu.VMEM((2, 128), jnp.int32),
            pltpu.VMEM((112, 128), jnp.int32),
            pltpu.SemaphoreType.DMA,
        ],
    )
    def _sc_gather(table_hbm, assign_hbm, out_hbm, tab_sh, tab_v, a_v, out_v,
                   sem):
        cid = lax.axis_index("c")
        sid = lax.axis_index("s")
        wid = sid * NC + cid

        with jax.named_scope("stage_table"):
            @pl.when(sid == 0)
            def _stage():
                pltpu.sync_copy(table_hbm, tab_sh)

            plsc.subcore_barrier()
            pltpu.sync_copy(tab_sh, tab_v)
            pltpu.sync_copy(assign_hbm.at[wid], a_v)

        lanes = lax.iota(jnp.int32, PS)
        kvec = lanes & 3

        def body(r, carry):
            for g in range(4):
                pvec = r * NROW + g * 4 + (lanes >> 2)
                a_lane = plsc.load_gather(a_v, [pvec >> 7, pvec & 127])
                trow = a_lane >> 1
                tbase = (a_lane & 1) * 64
                for i in range(PS):
                    words = plsc.load_gather(
                        tab_v, [trow, tbase + i * 4 + kvec])
                    out_v[8 * r + (4 * i + g) // 8,
                          pl.ds(64 * (i % 2) + 16 * g, PS)] = words
            return carry

        with jax.named_scope("assemble"):
            lax.fori_loop(0, NROW, body, 0)
        with jax.named_scope("writeback"):
            pltpu.sync_copy(out_v, out_hbm.at[wid])

    return _sc_gather


def kernel(image, centroids, cluster_labels):
    assign = _assignment(image, centroids)
    y4 = jnp.transpose(cluster_labels).reshape(K, 64, 4)
    table32 = (y4[..., 0] | (y4[..., 1] << 8)
               | (y4[..., 2] << 16) | (y4[..., 3] << 24))
    table32 = table32.reshape(K // 2, 128)
    out32 = _sc_gather_kernel()(table32, assign)
    words = out32.reshape(BS, IMG, 64)[:, :, :56]
    shifts = jnp.arange(4, dtype=jnp.int32) * 8
    pred = ((words[..., None] >> shifts) & 255).reshape(BS, IMG, IMG)
    return pred
